# baseline (speedup 1.0000x reference)
import functools

import jax
import jax.numpy as jnp
from jax import lax
from jax.experimental import pallas as pl
from jax.experimental.pallas import tpu as pltpu
from jax.experimental.pallas import tpu_sc as plsc

D1 = 48
D2 = 48
JC = 8


def _tc_body(cost_ref, out_ref, exp_ref):
    ht, w = cost_ref.shape[2], cost_ref.shape[3]
    exp_ref[...] = jnp.exp(cost_ref[0:3])
    for c0 in range(0, D2, JC):
        shp = (JC, ht, w)
        neg = jnp.full(shp, -jnp.inf, jnp.float32)
        zero_i = jnp.zeros(shp, jnp.int32)
        v0 = v1 = v2 = neg
        i0 = i1 = i2 = zero_i
        for i in range(D1):
            x = cost_ref[i, c0:c0 + JC]
            ix = jnp.full(shp, i, jnp.int32)
            b0 = x >= v0
            nv0 = jnp.maximum(v0, x)
            dx = jnp.minimum(v0, x)
            ni0 = jnp.where(b0, ix, i0)
            di = jnp.where(b0, i0, ix)
            b1 = dx >= v1
            nv1 = jnp.maximum(v1, dx)
            dx2 = jnp.minimum(v1, dx)
            ni1 = jnp.where(b1, di, i1)
            di2 = jnp.where(b1, i1, di)
            b2 = dx2 >= v2
            v2 = jnp.maximum(v2, dx2)
            i2 = jnp.where(b2, di2, i2)
            v0, v1, i0, i1 = nv0, nv1, ni0, ni1
        out_ref[c0:c0 + JC] = i0 | (i1 << 6) | (i2 << 12)


def _tc_scan(c):
    d1, d2, h, w = c.shape
    ht = 8
    return pl.pallas_call(
        _tc_body,
        grid=(h // ht,),
        in_specs=[pl.BlockSpec((d1, d2, ht, w), lambda g: (0, 0, g, 0))],
        out_specs=[
            pl.BlockSpec((d2, ht, w), lambda g: (0, g, 0)),
            pl.BlockSpec((3, d2, ht, w), lambda g: (0, 0, g, 0)),
        ],
        out_shape=[
            jax.ShapeDtypeStruct((d2, h, w), jnp.int32),
            jax.ShapeDtypeStruct((3, d2, h, w), jnp.float32),
        ],
        compiler_params=pltpu.CompilerParams(
            dimension_semantics=("parallel",)),
    )(c)


def _sc_kernel(npix, n_per_w, nw):
    mesh = plsc.VectorSubcoreMesh(core_axis_name="c", subcore_axis_name="s")

    @functools.partial(
        pl.kernel, mesh=mesh,
        out_type=jax.ShapeDtypeStruct((D2, npix), jnp.float32),
        scratch_types=[
            pltpu.VMEM((3 * D2, n_per_w), jnp.float32),
            pltpu.VMEM((D2, n_per_w), jnp.int32),
            pltpu.VMEM((D2, n_per_w), jnp.float32),
            pltpu.SemaphoreType.DMA,
        ],
        compiler_params=pltpu.CompilerParams(needs_layout_passes=False),
    )
    def k(rows_hbm, pidx_hbm, out_hbm, tab_v, p_v, o_v, sem):
        wid = lax.axis_index("s") * 2 + lax.axis_index("c")
        base = wid * n_per_w
        handles = []
        for i in range(3):
            handles.append(pltpu.async_copy(
                rows_hbm.at[i, :, pl.ds(base, n_per_w)],
                tab_v.at[pl.ds(i * D2, D2)], sem))
        handles.append(pltpu.async_copy(
            pidx_hbm.at[:, pl.ds(base, n_per_w)], p_v, sem))
        for hd in handles:
            hd.wait()

        lane = lax.iota(jnp.int32, 16)
        tpj = n_per_w // 16

        @plsc.parallel_loop(0, D2, unroll=2)
        def _loop(j):
            for t in range(tpj):
                pix = t * 16 + lane
                pp = p_v[j, pl.ds(t * 16, 16)]
                p0 = pp & 63
                p1 = (pp >> 6) & 63
                p2 = (pp >> 12) & 63
                e0 = plsc.load_gather(tab_v, [p0, pix])
                e1 = plsc.load_gather(tab_v, [p1 + D2, pix])
                e2 = plsc.load_gather(tab_v, [p2 + 2 * D2, pix])
                num = (e0 * p0.astype(jnp.float32)
                       + e1 * p1.astype(jnp.float32)
                       + e2 * p2.astype(jnp.float32))
                o_v[j, pl.ds(t * 16, 16)] = num / (e0 + e1 + e2)

        pltpu.async_copy(o_v, out_hbm.at[:, pl.ds(base, n_per_w)], sem).wait()

    return k


@jax.jit
def _run(cost):
    b, d1, d2, h, w = cost.shape
    c = cost.reshape(d1, d2, h, w)
    nchunk = 2
    hc = h // nchunk
    npix = hc * w
    nw = 32
    n_per_w = npix // nw
    sc = _sc_kernel(npix, n_per_w, nw)
    outs = []
    for k in range(nchunk):
        ck = jax.lax.slice_in_dim(c, k * hc, (k + 1) * hc, axis=2)
        pidx, erows = _tc_scan(ck)
        outs.append(sc(erows.reshape(3, d2, npix),
                       pidx.reshape(d2, npix)).reshape(d2, hc, w))
    out = jnp.concatenate(outs, axis=1)
    return out.reshape(b, 1, d2, h, w)


def kernel(cost):
    return _run(cost)


# speedup vs baseline: 1.5890x; 1.5890x over previous
import functools

import jax
import jax.numpy as jnp
from jax import lax
from jax.experimental import pallas as pl
from jax.experimental.pallas import tpu as pltpu
from jax.experimental.pallas import tpu_sc as plsc

D1 = 48
D2 = 48
JC = 8


def _tc_body(cost_ref, out_ref, exp_ref):
    ht, w = cost_ref.shape[2], cost_ref.shape[3]
    exp_ref[...] = jnp.exp(cost_ref[0:3])
    for c0 in range(0, D2, JC):
        shp = (JC, ht, w)
        neg = jnp.full(shp, -jnp.inf, jnp.float32)
        zero_i = jnp.zeros(shp, jnp.int32)
        v0 = v1 = v2 = neg
        i0 = i1 = i2 = zero_i
        for i in range(D1):
            x = cost_ref[i, c0:c0 + JC]
            ix = jnp.full(shp, i, jnp.int32)
            b0 = x >= v0
            nv0 = jnp.maximum(v0, x)
            dx = jnp.minimum(v0, x)
            ni0 = jnp.where(b0, ix, i0)
            di = jnp.where(b0, i0, ix)
            b1 = dx >= v1
            nv1 = jnp.maximum(v1, dx)
            dx2 = jnp.minimum(v1, dx)
            ni1 = jnp.where(b1, di, i1)
            di2 = jnp.where(b1, i1, di)
            b2 = dx2 >= v2
            v2 = jnp.maximum(v2, dx2)
            i2 = jnp.where(b2, di2, i2)
            v0, v1, i0, i1 = nv0, nv1, ni0, ni1
        out_ref[c0:c0 + JC] = i0 | (i1 << 6) | (i2 << 12)


def _tc_scan(c):
    d1, d2, h, w = c.shape
    ht = 8
    return pl.pallas_call(
        _tc_body,
        grid=(h // ht,),
        in_specs=[pl.BlockSpec((d1, d2, ht, w), lambda g: (0, 0, g, 0))],
        out_specs=[
            pl.BlockSpec((d2, ht, w), lambda g: (0, g, 0)),
            pl.BlockSpec((3, d2, ht, w), lambda g: (0, 0, g, 0)),
        ],
        out_shape=[
            jax.ShapeDtypeStruct((d2, h, w), jnp.int32),
            jax.ShapeDtypeStruct((3, d2, h, w), jnp.float32),
        ],
        compiler_params=pltpu.CompilerParams(
            dimension_semantics=("parallel",)),
    )(c)


def _sc_kernel(npix, n_per_w, nw):
    mesh = plsc.VectorSubcoreMesh(core_axis_name="c", subcore_axis_name="s")

    @functools.partial(
        pl.kernel, mesh=mesh,
        out_type=jax.ShapeDtypeStruct((D2, npix), jnp.float32),
        scratch_types=[
            pltpu.VMEM((3 * D2, n_per_w), jnp.float32),
            pltpu.VMEM((D2, n_per_w), jnp.int32),
            pltpu.VMEM((D2, n_per_w), jnp.float32),
            pltpu.SemaphoreType.DMA,
        ],
        compiler_params=pltpu.CompilerParams(needs_layout_passes=False),
    )
    def k(rows_hbm, pidx_hbm, out_hbm, tab_v, p_v, o_v, sem):
        wid = lax.axis_index("s") * 2 + lax.axis_index("c")
        base = wid * n_per_w
        handles = []
        for i in range(3):
            handles.append(pltpu.async_copy(
                rows_hbm.at[i, :, pl.ds(base, n_per_w)],
                tab_v.at[pl.ds(i * D2, D2)], sem))
        handles.append(pltpu.async_copy(
            pidx_hbm.at[:, pl.ds(base, n_per_w)], p_v, sem))
        for hd in handles:
            hd.wait()

        lane = lax.iota(jnp.int32, 16)
        tpj = n_per_w // 16

        @plsc.parallel_loop(0, D2, unroll=2)
        def _loop(j):
            for t in range(tpj):
                pix = t * 16 + lane
                pp = p_v[j, pl.ds(t * 16, 16)]
                p0 = pp & 63
                p1 = (pp >> 6) & 63
                p2 = (pp >> 12) & 63
                e0 = plsc.load_gather(tab_v, [p0, pix])
                e1 = plsc.load_gather(tab_v, [p1 + D2, pix])
                e2 = plsc.load_gather(tab_v, [p2 + 2 * D2, pix])
                num = (e0 * p0.astype(jnp.float32)
                       + e1 * p1.astype(jnp.float32)
                       + e2 * p2.astype(jnp.float32))
                o_v[j, pl.ds(t * 16, 16)] = num / (e0 + e1 + e2)

        pltpu.async_copy(o_v, out_hbm.at[:, pl.ds(base, n_per_w)], sem).wait()

    return k


@jax.jit
def _run(cost):
    b, d1, d2, h, w = cost.shape
    c = cost.reshape(d1, d2, h, w)
    pidx, erows = _tc_scan(c)
    npix = h * w
    nw = 32
    n_per_w = npix // nw
    out = _sc_kernel(npix, n_per_w, nw)(
        erows.reshape(3, d2, npix), pidx.reshape(d2, npix))
    return out.reshape(b, 1, d2, h, w)


def kernel(cost):
    return _run(cost)
